# Initial kernel scaffold; baseline (speedup 1.0000x reference)
import jax
import jax.numpy as jnp
from jax.experimental import pallas as pl


def kernel(species, aev, atom_index12, distances, total_charges, W0, b0, Wn0, bn0, W1, b1, Wn1, bn1, Wf, bf, decay_prefactor, decay_factor):
    raise NotImplementedError("write your pallas kernel here")


# speedup vs baseline: 3.1940x; 3.1940x over previous
import functools

import jax
import jax.numpy as jnp
from jax import lax
from jax.experimental import pallas as pl
from jax.experimental.pallas import tpu as pltpu
from jax.experimental.pallas import tpu_sc as plsc

RC = 5.2

NC, NS, L = 2, 16, 16
NW = NC * NS

CHUNK = 128

MROWS = 10240
RPT = MROWS // NS


def _celu(x):
    return jnp.where(x > 0, x, jnp.exp(x) - 1.0)


def _expert_linear(x, oh, w_ref, b_ref):
    n_sp = w_ref.shape[0]
    acc = None
    for s in range(n_sp):
        y = jnp.dot(x, w_ref[s], preferred_element_type=jnp.float32)
        y = oh[:, s:s + 1] * (y + b_ref[s][None, :])
        acc = y if acc is None else acc + y
    return acc


def _k1_body(x_ref, oh_ref, w0_ref, b0_ref, wn0_ref, bn0_ref,
             int_ref, ngh_ref):
    x = x_ref[...]
    oh = oh_ref[...]
    internal = _celu(_expert_linear(x, oh, w0_ref, b0_ref))
    int_ref[...] = internal
    ngh_ref[...] = _celu(_expert_linear(internal, oh, wn0_ref, bn0_ref))


def _k2_body(x_ref, ma_ref, mb_ref, oh_ref, w1a_ref, w1b_ref, b1_ref,
             wn1_ref, bn1_ref, int_ref, ngh_ref):
    x = x_ref[...]
    m = ma_ref[...] + mb_ref[...]
    oh = oh_ref[...]
    n_sp = w1a_ref.shape[0]
    acc = None
    for s in range(n_sp):
        y = jnp.dot(x, w1a_ref[s], preferred_element_type=jnp.float32)
        y = y + jnp.dot(m, w1b_ref[s], preferred_element_type=jnp.float32)
        y = oh[:, s:s + 1] * (y + b1_ref[s][None, :])
        acc = y if acc is None else acc + y
    internal = _celu(acc)
    int_ref[...] = internal
    ngh_ref[...] = _celu(_expert_linear(internal, oh, wn1_ref, bn1_ref))


def _k3_body(x_ref, ma_ref, mb_ref, oh_ref, sp_ref, wfa_ref, wfb_ref,
             bf_ref, tq_ref, ch_ref, pc_ref):
    x = x_ref[...]
    m = ma_ref[...] + mb_ref[...]
    p = jnp.dot(x, wfa_ref[...], preferred_element_type=jnp.float32)
    p = p + jnp.dot(m, wfb_ref[...], preferred_element_type=jnp.float32)
    p = p + bf_ref[...]
    prech = jnp.sum(oh_ref[...] * p, axis=1)
    b, a = sp_ref.shape
    prech = prech.reshape(b, a)
    sp = sp_ref[...]
    dummy = sp != -1
    cnt = jnp.sum(dummy.astype(jnp.float32), axis=1, keepdims=True)
    tp = jnp.sum(prech, axis=1, keepdims=True)
    ch = prech + (tq_ref[...] - tp) / cnt
    ch_ref[...] = jnp.where(dummy, ch, 0.0)
    pc_ref[...] = prech


def _sc_scatter_build(cpt):
    mesh = plsc.VectorSubcoreMesh(core_axis_name="c", subcore_axis_name="s",
                                  num_cores=NC, num_subcores=NS)

    @functools.partial(
        pl.kernel,
        out_type=jax.ShapeDtypeStruct((NC, MROWS, 128), jnp.float32),
        mesh=mesh,
        scratch_types=[
            pltpu.VMEM((cpt, CHUNK), jnp.int32),
            pltpu.VMEM((cpt, CHUNK), jnp.int32),
            pltpu.VMEM((cpt, CHUNK), jnp.float32),
            pltpu.VMEM((CHUNK, 128), jnp.float32),
            pltpu.VMEM((CHUNK,), jnp.float32),
            pltpu.VMEM((2, L), jnp.float32),
            pltpu.VMEM_SHARED((MROWS, 128), jnp.float32),
            pltpu.SemaphoreType.DMA,
        ],
        compiler_params=pltpu.CompilerParams(needs_layout_passes=False),
    )
    def sc_kernel(neigh, isrc, idst, dist, dcoef, out,
                  isrc_v, idst_v, dist_v, rows_v, dec_v, dcoef_v, acc, sem):
        c = lax.axis_index("c")
        s = lax.axis_index("s")
        w = c * NS + s

        def _zrow(r, carry):
            for g in range(8):
                rows_v[r, pl.ds(g * L, L)] = jnp.zeros((L,), jnp.float32)
            return carry
        lax.fori_loop(0, CHUNK, _zrow, 0)
        for k in range(RPT // CHUNK):
            pltpu.sync_copy(rows_v,
                            acc.at[pl.ds(s * RPT + k * CHUNK, CHUNK)])

        pltpu.sync_copy(isrc.at[pl.ds(w * cpt, cpt)], isrc_v)
        pltpu.sync_copy(idst.at[pl.ds(w * cpt, cpt)], idst_v)
        pltpu.sync_copy(dist.at[pl.ds(w * cpt, cpt)], dist_v)
        pltpu.sync_copy(dcoef, dcoef_v)
        dp2 = dcoef_v[0, :]
        df2 = dcoef_v[1, :]

        plsc.subcore_barrier()

        def _chunk(j, carry):
            pltpu.async_copy(neigh.at[isrc_v.at[j]], rows_v, sem).wait()
            for g in range(8):
                d = dist_v[j, pl.ds(g * L, L)]
                x = d * (1.0 / RC)
                x2 = jnp.clip(x * x, 0.0, 1.0 - 1e-6)
                f = jnp.exp(1.0 - 1.0 / (1.0 - x2))
                dec = jnp.where(d < RC, f, 0.0)
                dec = dp2 * jnp.exp(-df2 * d) * dec
                dec_v[pl.ds(g * L, L)] = dec
            def _scale8(t, carry2):
                for u in range(8):
                    e = t * 8 + u
                    bc = plsc.load_gather(
                        dec_v, [jnp.full((L,), e, jnp.int32)])
                    for q in range(8):
                        rows_v[e, pl.ds(q * L, L)] = (
                            rows_v[e, pl.ds(q * L, L)] * bc)
                return carry2
            lax.fori_loop(0, CHUNK // 8, _scale8, 0)
            pltpu.sync_copy(rows_v, acc.at[idst_v.at[j]], add=True)
            return carry
        lax.fori_loop(0, cpt, _chunk, 0)

        plsc.subcore_barrier()
        pltpu.sync_copy(acc.at[pl.ds(s * RPT, RPT)],
                        out.at[c, pl.ds(s * RPT, RPT)])

    return sc_kernel


def kernel(species, aev, atom_index12, distances, total_charges,
           W0, b0, Wn0, bn0, W1, b1, Wn1, bn1, Wf, bf,
           decay_prefactor, decay_factor):
    bsz, na = species.shape
    n = bsz * na
    d_aev = aev.shape[-1]
    n_sp = W0.shape[0]
    m0 = W0.shape[-1]
    nb0 = Wn0.shape[-1]
    m1 = W1.shape[-1]
    nb1 = Wn1.shape[-1]
    p = atom_index12.shape[1]

    species_ = species.reshape(-1)
    feats = aev.reshape(n, d_aev)
    oh = (species_[:, None] == jnp.arange(n_sp, dtype=species_.dtype)[None, :]
          ).astype(jnp.float32)

    idx_dst = atom_index12.reshape(-1).astype(jnp.int32)
    idx_src = atom_index12[::-1].reshape(-1).astype(jnp.int32)
    dist2 = jnp.concatenate([distances, distances])
    p2 = 2 * p
    ept = 8 * CHUNK * -(-p2 // (8 * CHUNK * NW))
    cpt = ept // CHUNK
    pad = NW * ept - p2
    idx_src_p = jnp.concatenate(
        [idx_src, jnp.zeros((pad,), jnp.int32)]).reshape(-1, CHUNK)
    idx_dst_p = jnp.concatenate(
        [idx_dst, jnp.zeros((pad,), jnp.int32)]).reshape(-1, CHUNK)
    dist_p = jnp.concatenate(
        [dist2, jnp.full((pad,), 1e9, jnp.float32)]).reshape(-1, CHUNK)
    dcoef = jnp.stack([
        jnp.full((L,), decay_prefactor.astype(jnp.float32) ** 2),
        jnp.full((L,), decay_factor.astype(jnp.float32) ** 2)])

    sc_scatter = _sc_scatter_build(cpt)

    rows = 400
    grid = (n // rows,)
    wspec3 = lambda shp: pl.BlockSpec(shp, lambda i: (0, 0, 0))
    wspec2 = lambda shp: pl.BlockSpec(shp, lambda i: (0, 0))
    rspec = lambda width: pl.BlockSpec((rows, width), lambda i: (i, 0))

    k1 = pl.pallas_call(
        _k1_body,
        grid=grid,
        in_specs=[rspec(d_aev), rspec(n_sp),
                  wspec3((n_sp, d_aev, m0)), wspec2((n_sp, m0)),
                  wspec3((n_sp, m0, nb0)), wspec2((n_sp, nb0))],
        out_specs=[rspec(m0), rspec(nb0)],
        out_shape=[jax.ShapeDtypeStruct((n, m0), jnp.float32),
                   jax.ShapeDtypeStruct((n, nb0), jnp.float32)],
    )
    internal0, neigh0 = k1(feats, oh, W0, b0, Wn0, bn0)

    merged0 = sc_scatter(neigh0, idx_src_p, idx_dst_p, dist_p, dcoef)

    W1a = W1[:, :m0, :]
    W1b = W1[:, m0:, :]
    mspec = pl.BlockSpec((rows, nb0), lambda i: (i, 0))
    k2 = pl.pallas_call(
        _k2_body,
        grid=grid,
        in_specs=[rspec(m0), mspec, mspec, rspec(n_sp),
                  wspec3((n_sp, m0, m1)), wspec3((n_sp, nb0, m1)),
                  wspec2((n_sp, m1)),
                  wspec3((n_sp, m1, nb1)), wspec2((n_sp, nb1))],
        out_specs=[rspec(m1), rspec(nb1)],
        out_shape=[jax.ShapeDtypeStruct((n, m1), jnp.float32),
                   jax.ShapeDtypeStruct((n, nb1), jnp.float32)],
    )
    internal1, neigh1 = k2(internal0, merged0[0], merged0[1], oh,
                           W1a, W1b, b1, Wn1, bn1)

    merged1 = sc_scatter(neigh1, idx_src_p, idx_dst_p, dist_p, dcoef)

    wfa = Wf[:, :m1, 0].T
    wfb = Wf[:, m1:, 0].T
    bfv = bf.reshape(1, n_sp)
    z2 = lambda i: (0, 0)
    k3 = pl.pallas_call(
        _k3_body,
        grid=(1,),
        in_specs=[pl.BlockSpec((n, m1), z2),
                  pl.BlockSpec((n, nb1), z2),
                  pl.BlockSpec((n, nb1), z2),
                  pl.BlockSpec((n, n_sp), z2),
                  pl.BlockSpec((bsz, na), z2),
                  pl.BlockSpec((m1, n_sp), z2),
                  pl.BlockSpec((nb1, n_sp), z2),
                  pl.BlockSpec((1, n_sp), z2),
                  pl.BlockSpec((bsz, 1), z2)],
        out_specs=[pl.BlockSpec((bsz, na), z2),
                   pl.BlockSpec((bsz, na), z2)],
        out_shape=[jax.ShapeDtypeStruct((bsz, na), jnp.float32),
                   jax.ShapeDtypeStruct((bsz, na), jnp.float32)],
    )
    charges, precharges = k3(internal1, merged1[0], merged1[1], oh, species,
                             wfa, wfb, bfv, total_charges.reshape(bsz, 1))
    return species, charges, precharges
